# baseline (speedup 1.0000x reference)
import functools

import jax
import jax.numpy as jnp
import numpy as np
from jax import lax
from jax.experimental import pallas as pl
from jax.experimental.pallas import tpu as pltpu
from jax.experimental.pallas import tpu_sc as plsc

BATCH = 4096
NOUT = 4096
NLANE = 16
NW = 32
ROWS_PER_W = BATCH // NW
R = 4
NBLK = ROWS_PER_W // R
NCH = NOUT // NLANE

_M = np.zeros((8, 16), np.float32)
for _i in (8, 9, 10, 11, 12, 13, 14, 15):
    _M[0, _i] = 1.0
for _i, _v in ((2, 1), (3, 1), (6, 1), (7, 1), (8, -1), (9, -1), (12, -1), (13, -1)):
    _M[1, _i] = _v
for _i, _v in ((4, 1), (5, 1), (6, 1), (7, 1), (8, -1), (9, -1), (10, -1), (11, -1)):
    _M[2, _i] = _v
for _i, _v in ((1, 1), (2, -1), (4, -1), (6, -2), (7, -1), (8, 1), (9, 2),
               (11, 1), (13, 1), (14, -1)):
    _M[3, _i] = _v


def _coef_body(m_ref, w_ref, o_ref):
    w = w_ref[...]
    m = jnp.max(w, axis=1, keepdims=True)
    e = jnp.exp(w - m)
    p = e / jnp.sum(e, axis=1, keepdims=True)
    o_ref[...] = lax.dot_general(
        m_ref[...], p, (((1,), (1,)), ((), ())),
        preferred_element_type=jnp.float32)


_coef_call = pl.pallas_call(
    _coef_body,
    out_shape=jax.ShapeDtypeStruct((8, NOUT), jnp.float32),
)


def _sc_body(x_hbm, idx_hbm, coef_hbm, out_hbm, idx_v, coef_v, xb, ob):
    mesh_nc = 2
    wid = lax.axis_index("s") * mesh_nc + lax.axis_index("c")
    base = wid * ROWS_PER_W
    pltpu.sync_copy(idx_hbm, idx_v)
    pltpu.sync_copy(coef_hbm.at[pl.ds(0, 4 * NOUT)], coef_v)

    def blk_body(blk, carry):
        row0 = base + blk * R
        pltpu.sync_copy(x_hbm.at[pl.ds(row0 * NOUT, R * NOUT)], xb)

        def ch_body(c, inner):
            s = c * NLANE
            ia = idx_v[pl.ds(s, NLANE)]
            ib = idx_v[pl.ds(NOUT + s, NLANE)]
            c0 = coef_v[pl.ds(s, NLANE)]
            ca = coef_v[pl.ds(NOUT + s, NLANE)]
            cb = coef_v[pl.ds(2 * NOUT + s, NLANE)]
            cab = coef_v[pl.ds(3 * NOUT + s, NLANE)]
            for r in range(R):
                a = plsc.load_gather(xb, [ia + (r * NOUT)])
                b = plsc.load_gather(xb, [ib + (r * NOUT)])
                f1 = c0 + ca * a
                f2 = cb + cab * a
                ob[pl.ds(r * NOUT + s, NLANE)] = f1 + b * f2
            return inner

        lax.fori_loop(0, NCH, ch_body, 0)
        pltpu.sync_copy(ob, out_hbm.at[pl.ds(row0 * NOUT, R * NOUT)])
        return carry

    lax.fori_loop(0, NBLK, blk_body, 0)


_sc_call = pl.kernel(
    _sc_body,
    out_type=jax.ShapeDtypeStruct((BATCH * NOUT,), jnp.float32),
    mesh=plsc.VectorSubcoreMesh(core_axis_name="c", subcore_axis_name="s"),
    compiler_params=pltpu.CompilerParams(needs_layout_passes=False),
    scratch_types=[
        pltpu.VMEM((2 * NOUT,), jnp.int32),
        pltpu.VMEM((4 * NOUT,), jnp.float32),
        pltpu.VMEM((R * NOUT,), jnp.float32),
        pltpu.VMEM((R * NOUT,), jnp.float32),
    ],
)


def kernel(x, weights, idx_a, idx_b):
    idx2 = jnp.concatenate(
        [idx_a.astype(jnp.int32), idx_b.astype(jnp.int32)])
    coef = _coef_call(jnp.asarray(_M), weights).reshape(-1)
    out = _sc_call(x.reshape(-1), idx2, coef)
    return out.reshape(BATCH, NOUT)


# speedup vs baseline: 1.1170x; 1.1170x over previous
import functools

import jax
import jax.numpy as jnp
import numpy as np
from jax import lax
from jax.experimental import pallas as pl
from jax.experimental.pallas import tpu as pltpu
from jax.experimental.pallas import tpu_sc as plsc

BATCH = 4096
NOUT = 4096
NLANE = 16
NW = 32
ROWS_PER_W = BATCH // NW
R = 4
NBLK = ROWS_PER_W // R
NCH = NOUT // NLANE

_M = np.zeros((8, 16), np.float32)
for _i in (8, 9, 10, 11, 12, 13, 14, 15):
    _M[0, _i] = 1.0
for _i, _v in ((2, 1), (3, 1), (6, 1), (7, 1), (8, -1), (9, -1), (12, -1), (13, -1)):
    _M[1, _i] = _v
for _i, _v in ((4, 1), (5, 1), (6, 1), (7, 1), (8, -1), (9, -1), (10, -1), (11, -1)):
    _M[2, _i] = _v
for _i, _v in ((1, 1), (2, -1), (4, -1), (6, -2), (7, -1), (8, 1), (9, 2),
               (11, 1), (13, 1), (14, -1)):
    _M[3, _i] = _v


def _coef_body(m_ref, w_ref, o_ref):
    w = w_ref[...]
    m = jnp.max(w, axis=1, keepdims=True)
    e = jnp.exp(w - m)
    p = e / jnp.sum(e, axis=1, keepdims=True)
    o_ref[...] = lax.dot_general(
        m_ref[...], p, (((1,), (1,)), ((), ())),
        preferred_element_type=jnp.float32)


_coef_call = pl.pallas_call(
    _coef_body,
    out_shape=jax.ShapeDtypeStruct((8, NOUT), jnp.float32),
)


def _sc_body(x_hbm, idx_hbm, coef_hbm, out_hbm, idx_v, coef_v, xb, ob):
    mesh_nc = 2
    wid = lax.axis_index("s") * mesh_nc + lax.axis_index("c")
    base = wid * ROWS_PER_W
    pltpu.sync_copy(idx_hbm, idx_v)
    pltpu.sync_copy(coef_hbm.at[pl.ds(0, 4), :], coef_v)

    def blk_body(blk, carry):
        row0 = base + blk * R
        pltpu.sync_copy(x_hbm.at[pl.ds(row0, R), :], xb)

        def ch_body(c, inner):
            s = c * NLANE
            ia = idx_v[0, pl.ds(s, NLANE)]
            ib = idx_v[1, pl.ds(s, NLANE)]
            c0 = coef_v[0, pl.ds(s, NLANE)]
            ca = coef_v[1, pl.ds(s, NLANE)]
            cb = coef_v[2, pl.ds(s, NLANE)]
            cab = coef_v[3, pl.ds(s, NLANE)]
            for r in range(R):
                ridx = jnp.full((NLANE,), r, jnp.int32)
                a = plsc.load_gather(xb, [ridx, ia])
                b = plsc.load_gather(xb, [ridx, ib])
                f1 = c0 + ca * a
                f2 = cb + cab * a
                ob[r, pl.ds(s, NLANE)] = f1 + b * f2
            return inner

        lax.fori_loop(0, NCH, ch_body, 0)
        pltpu.sync_copy(ob, out_hbm.at[pl.ds(row0, R), :])
        return carry

    lax.fori_loop(0, NBLK, blk_body, 0)


_sc_call = pl.kernel(
    _sc_body,
    out_type=jax.ShapeDtypeStruct((BATCH, NOUT), jnp.float32),
    mesh=plsc.VectorSubcoreMesh(core_axis_name="c", subcore_axis_name="s"),
    compiler_params=pltpu.CompilerParams(needs_layout_passes=False),
    scratch_types=[
        pltpu.VMEM((2, NOUT), jnp.int32),
        pltpu.VMEM((4, NOUT), jnp.float32),
        pltpu.VMEM((R, NOUT), jnp.float32),
        pltpu.VMEM((R, NOUT), jnp.float32),
    ],
)


def kernel(x, weights, idx_a, idx_b):
    idx2 = jnp.stack([idx_a.astype(jnp.int32), idx_b.astype(jnp.int32)])
    coef = _coef_call(jnp.asarray(_M), weights)
    return _sc_call(x, idx2, coef)


# speedup vs baseline: 4.4126x; 3.9504x over previous
import functools

import jax
import jax.numpy as jnp
import numpy as np
from jax import lax
from jax.experimental import pallas as pl
from jax.experimental.pallas import tpu as pltpu
from jax.experimental.pallas import tpu_sc as plsc

BATCH = 4096
NOUT = 4096
NLANE = 16
NW = 32
ROWS_PER_W = BATCH // NW
R = 8
NBLK = ROWS_PER_W // R
GW = 512
NGRP = NOUT // GW
CPG = GW // NLANE

_M = np.zeros((8, 16), np.float32)
for _i in (8, 9, 10, 11, 12, 13, 14, 15):
    _M[0, _i] = 1.0
for _i, _v in ((2, 1), (3, 1), (6, 1), (7, 1), (8, -1), (9, -1), (12, -1), (13, -1)):
    _M[1, _i] = _v
for _i, _v in ((4, 1), (5, 1), (6, 1), (7, 1), (8, -1), (9, -1), (10, -1), (11, -1)):
    _M[2, _i] = _v
for _i, _v in ((1, 1), (2, -1), (4, -1), (6, -2), (7, -1), (8, 1), (9, 2),
               (11, 1), (13, 1), (14, -1)):
    _M[3, _i] = _v


def _coef_body(m_ref, w_ref, o_ref):
    w = w_ref[...]
    m = jnp.max(w, axis=1, keepdims=True)
    e = jnp.exp(w - m)
    p = e / jnp.sum(e, axis=1, keepdims=True)
    o_ref[...] = lax.dot_general(
        m_ref[...], p, (((1,), (1,)), ((), ())),
        preferred_element_type=jnp.float32)


_coef_call = pl.pallas_call(
    _coef_body,
    out_shape=jax.ShapeDtypeStruct((8, NOUT), jnp.float32),
)


def _sc_body(x_hbm, idx_hbm, coef_hbm, out_hbm,
             idx_v, coef_v, xb0, xb1, og0, og1,
             sx0, sx1, so0, so1):
    mesh_nc = 2
    wid = lax.axis_index("s") * mesh_nc + lax.axis_index("c")
    base = wid * ROWS_PER_W
    pltpu.sync_copy(idx_hbm, idx_v)
    pltpu.sync_copy(coef_hbm.at[pl.ds(0, 4), :], coef_v)

    xbufs = (xb0, xb1)
    xsems = (sx0, sx1)
    obufs = (og0, og1)
    osems = (so0, so1)

    def x_src(blk):
        return x_hbm.at[pl.ds(base + blk * R, R), :]

    pltpu.async_copy(x_src(0), xb0, sx0)
    pltpu.async_copy(x_src(1), xb1, sx1)

    def blk_pair(i2, carry):
        for b in (0, 1):
            blk = i2 * 2 + b
            row0 = base + blk * R
            xb = xbufs[b]
            pltpu.make_async_copy(x_src(blk), xb, xsems[b]).wait()

            def grp_pair(g2, inner):
                for h in (0, 1):
                    g = g2 * 2 + h
                    og = obufs[h]
                    dst = out_hbm.at[pl.ds(row0, R), pl.ds(g * GW, GW)]

                    @pl.when(blk * NGRP + g >= 2)
                    def _():
                        pltpu.make_async_copy(og, dst, osems[h]).wait()

                    @plsc.parallel_loop(0, CPG, unroll=2)
                    def chunk_body(cc):
                        s = g * GW + cc * NLANE
                        ia = idx_v[0, pl.ds(s, NLANE)]
                        ib = idx_v[1, pl.ds(s, NLANE)]
                        c0 = coef_v[0, pl.ds(s, NLANE)]
                        ca = coef_v[1, pl.ds(s, NLANE)]
                        cb = coef_v[2, pl.ds(s, NLANE)]
                        cab = coef_v[3, pl.ds(s, NLANE)]
                        avs = []
                        bvs = []
                        for r in range(R):
                            ridx = jnp.full((NLANE,), r, jnp.int32)
                            avs.append(plsc.load_gather(xb, [ridx, ia]))
                            bvs.append(plsc.load_gather(xb, [ridx, ib]))
                        for r in range(R):
                            f1 = c0 + ca * avs[r]
                            f2 = cb + cab * avs[r]
                            og[r, pl.ds(cc * NLANE, NLANE)] = f1 + bvs[r] * f2

                    pltpu.async_copy(og, dst, osems[h])
                return inner

            lax.fori_loop(0, NGRP // 2, grp_pair, 0)

            @pl.when(blk < NBLK - 2)
            def _():
                pltpu.async_copy(x_src(blk + 2), xb, xsems[b])
        return carry

    lax.fori_loop(0, NBLK // 2, blk_pair, 0)

    last0 = out_hbm.at[pl.ds(base + (NBLK - 1) * R, R),
                       pl.ds((NGRP - 2) * GW, GW)]
    last1 = out_hbm.at[pl.ds(base + (NBLK - 1) * R, R),
                       pl.ds((NGRP - 1) * GW, GW)]
    pltpu.make_async_copy(og0, last0, so0).wait()
    pltpu.make_async_copy(og1, last1, so1).wait()


_sc_call = pl.kernel(
    _sc_body,
    out_type=jax.ShapeDtypeStruct((BATCH, NOUT), jnp.float32),
    mesh=plsc.VectorSubcoreMesh(core_axis_name="c", subcore_axis_name="s"),
    compiler_params=pltpu.CompilerParams(needs_layout_passes=False),
    scratch_types=[
        pltpu.VMEM((2, NOUT), jnp.int32),
        pltpu.VMEM((4, NOUT), jnp.float32),
        pltpu.VMEM((R, NOUT), jnp.float32),
        pltpu.VMEM((R, NOUT), jnp.float32),
        pltpu.VMEM((R, GW), jnp.float32),
        pltpu.VMEM((R, GW), jnp.float32),
        pltpu.SemaphoreType.DMA,
        pltpu.SemaphoreType.DMA,
        pltpu.SemaphoreType.DMA,
        pltpu.SemaphoreType.DMA,
    ],
)


def kernel(x, weights, idx_a, idx_b):
    idx2 = jnp.stack([idx_a.astype(jnp.int32), idx_b.astype(jnp.int32)])
    coef = _coef_call(jnp.asarray(_M), weights)
    return _sc_call(x, idx2, coef)
